# baseline (speedup 1.0000x reference)
import functools
import math

import jax
import jax.numpy as jnp
from jax.experimental import pallas as pl
from jax.experimental.pallas import tpu as pltpu

_RATIO = 0.5
_FS = 4


def _cdiv(a, b):
    return (a + b - 1) // b


def _mm_body(a_ref, b_ref, o_ref, acc):
    @pl.when(pl.program_id(2) == 0)
    def _():
        acc[...] = jnp.zeros_like(acc)

    acc[...] += jnp.dot(a_ref[...], b_ref[...],
                        preferred_element_type=jnp.float32)

    @pl.when(pl.program_id(2) == pl.num_programs(2) - 1)
    def _():
        o_ref[...] = acc[...]


def _mm(a, b, bm=512, bk=512, bn=512):
    m, kk = a.shape
    _, n = b.shape
    bm = min(bm, m)
    bk = min(bk, kk)
    bn = min(bn, n)
    return pl.pallas_call(
        _mm_body,
        grid=(m // bm, n // bn, kk // bk),
        in_specs=[
            pl.BlockSpec((bm, bk), lambda i, j, k: (i, k)),
            pl.BlockSpec((bk, bn), lambda i, j, k: (k, j)),
        ],
        out_specs=pl.BlockSpec((bm, bn), lambda i, j, k: (i, j)),
        out_shape=jax.ShapeDtypeStruct((m, n), jnp.float32),
        scratch_shapes=[pltpu.VMEM((bm, bn), jnp.float32)],
        compiler_params=pltpu.CompilerParams(
            dimension_semantics=("parallel", "parallel", "arbitrary")),
    )(a, b)


def _total_body(c_ref, a_ref, b2_ref, b3_ref, b4_ref, tot_ref, nnz_ref, *, bm):
    i = pl.program_id(0)
    j = pl.program_id(1)
    c0 = c_ref[0, 0]
    c1 = c_ref[0, 1]
    c2 = c_ref[0, 2]
    c3 = c_ref[0, 3]
    c4 = c_ref[0, 4]
    t = c1 * a_ref[...] + c2 * b2_ref[...]
    t = t + c3 * b3_ref[...] + c4 * b4_ref[...]

    @pl.when(i == j)
    def _():
        rows = jax.lax.broadcasted_iota(jnp.int32, t.shape, 0)
        cols = jax.lax.broadcasted_iota(jnp.int32, t.shape, 1)
        tot_ref[...] = t + jnp.where(rows == cols, c0, 0.0)

    @pl.when(i != j)
    def _():
        tot_ref[...] = t

    cnt = jnp.sum((tot_ref[...] != 0.0).astype(jnp.float32), axis=1,
                  keepdims=True)

    @pl.when(j == 0)
    def _():
        nnz_ref[...] = cnt

    @pl.when(j != 0)
    def _():
        nnz_ref[...] += cnt


def _total_nnz(c_pad, a, b2, b3, b4, bm=512):
    n = a.shape[0]
    bm = min(bm, n)
    grid = (n // bm, n // bm)
    spec = pl.BlockSpec((bm, bm), lambda i, j: (i, j))
    return pl.pallas_call(
        functools.partial(_total_body, bm=bm),
        grid=grid,
        in_specs=[
            pl.BlockSpec((1, 128), lambda i, j: (0, 0)),
            spec, spec, spec, spec,
        ],
        out_specs=[
            pl.BlockSpec((bm, bm), lambda i, j: (i, j)),
            pl.BlockSpec((bm, 1), lambda i, j: (i, 0)),
        ],
        out_shape=[
            jax.ShapeDtypeStruct((n, n), jnp.float32),
            jax.ShapeDtypeStruct((n, 1), jnp.float32),
        ],
        compiler_params=pltpu.CompilerParams(
            dimension_semantics=("parallel", "arbitrary")),
    )(c_pad, a, b2, b3, b4)


def _m_body(tot_ref, dr_ref, dc_ref, m_ref, s2_ref):
    i = pl.program_id(1)
    m = tot_ref[...] * dr_ref[...] * dc_ref[...]
    m_ref[...] = m
    part = jnp.sum(m, axis=0, keepdims=True)

    @pl.when(i == 0)
    def _():
        s2_ref[...] = part

    @pl.when(i != 0)
    def _():
        s2_ref[...] += part


def _m_and_colsum(total, dis_r, dis_c, bm=512):
    n = total.shape[0]
    bm = min(bm, n)
    return pl.pallas_call(
        _m_body,
        grid=(n // bm, n // bm),
        in_specs=[
            pl.BlockSpec((bm, bm), lambda j, i: (i, j)),
            pl.BlockSpec((bm, 1), lambda j, i: (i, 0)),
            pl.BlockSpec((1, bm), lambda j, i: (0, j)),
        ],
        out_specs=[
            pl.BlockSpec((bm, bm), lambda j, i: (i, j)),
            pl.BlockSpec((1, bm), lambda j, i: (0, j)),
        ],
        out_shape=[
            jax.ShapeDtypeStruct((n, n), jnp.float32),
            jax.ShapeDtypeStruct((1, n), jnp.float32),
        ],
        compiler_params=pltpu.CompilerParams(
            dimension_semantics=("parallel", "arbitrary")),
    )(total, dis_r, dis_c)


def _xside_body(m_ref, x_ref, wt_ref, b_ref, p_ref, s2_ref, beta_ref,
                xs_ref, sc_ref, acc):
    k = pl.program_id(1)

    @pl.when(k == 0)
    def _():
        acc[...] = jnp.zeros_like(acc)

    acc[...] += jnp.dot(m_ref[...], x_ref[...],
                        preferred_element_type=jnp.float32)

    @pl.when(k == pl.num_programs(1) - 1)
    def _():
        lin = jnp.dot(acc[...], wt_ref[...],
                      preferred_element_type=jnp.float32) + b_ref[...]
        r = jnp.maximum(lin, 0.0)
        s1 = jnp.sum(r * p_ref[...], axis=1, keepdims=True)
        xs_ref[...] = r
        sc_ref[...] = beta_ref[0, 0] * s1 + beta_ref[0, 1] * s2_ref[...]


def _xside(m, x, wt, b2d, p2d, s2col, beta_pad, bm=512, bk=512):
    n = m.shape[0]
    d_in = x.shape[1]
    d_out = wt.shape[1]
    bm = min(bm, n)
    bk = min(bk, n)
    xs, sc = pl.pallas_call(
        _xside_body,
        grid=(n // bm, n // bk),
        in_specs=[
            pl.BlockSpec((bm, bk), lambda i, k: (i, k)),
            pl.BlockSpec((bk, d_in), lambda i, k: (k, 0)),
            pl.BlockSpec((d_in, d_out), lambda i, k: (0, 0)),
            pl.BlockSpec((1, d_out), lambda i, k: (0, 0)),
            pl.BlockSpec((1, d_out), lambda i, k: (0, 0)),
            pl.BlockSpec((bm, 1), lambda i, k: (i, 0)),
            pl.BlockSpec((1, 128), lambda i, k: (0, 0)),
        ],
        out_specs=[
            pl.BlockSpec((bm, d_out), lambda i, k: (i, 0)),
            pl.BlockSpec((bm, 1), lambda i, k: (i, 0)),
        ],
        out_shape=[
            jax.ShapeDtypeStruct((n, d_out), jnp.float32),
            jax.ShapeDtypeStruct((n, 1), jnp.float32),
        ],
        scratch_shapes=[pltpu.VMEM((bm, d_in), jnp.float32)],
        compiler_params=pltpu.CompilerParams(
            dimension_semantics=("parallel", "arbitrary")),
    )(m, x, wt, b2d, p2d, s2col, beta_pad)
    return xs, sc


def _scale_body(r_ref, sc_ref, o_ref):
    o_ref[...] = r_ref[...] * sc_ref[...]


def _scale_rows(r, score, bm=512):
    n, d = r.shape
    bm = min(bm, n)
    return pl.pallas_call(
        _scale_body,
        grid=(n // bm,),
        in_specs=[
            pl.BlockSpec((bm, d), lambda i: (i, 0)),
            pl.BlockSpec((bm, 1), lambda i: (i, 0)),
        ],
        out_specs=pl.BlockSpec((bm, d), lambda i: (i, 0)),
        out_shape=jax.ShapeDtypeStruct((n, d), jnp.float32),
    )(r, score)


def _rank_body(sr_ref, sc_ref, rank_ref, *, bm, bn):
    i = pl.program_id(0)
    j = pl.program_id(1)
    sr = sr_ref[...]
    sc = sc_ref[...]
    ig = i * bm + jax.lax.broadcasted_iota(jnp.int32, (bm, bn), 0)
    jg = j * bn + jax.lax.broadcasted_iota(jnp.int32, (bm, bn), 1)
    gt = (sc > sr)
    eq_lt = (sc == sr) & (jg < ig)
    contrib = jnp.sum((gt | eq_lt).astype(jnp.float32), axis=1, keepdims=True)

    @pl.when(j == 0)
    def _():
        rank_ref[...] = contrib

    @pl.when(j != 0)
    def _():
        rank_ref[...] += contrib


def _rank(score_col, score_row, bm=512):
    n = score_col.shape[0]
    bm = min(bm, n)
    return pl.pallas_call(
        functools.partial(_rank_body, bm=bm, bn=bm),
        grid=(n // bm, n // bm),
        in_specs=[
            pl.BlockSpec((bm, 1), lambda i, j: (i, 0)),
            pl.BlockSpec((1, bm), lambda i, j: (0, j)),
        ],
        out_specs=pl.BlockSpec((bm, 1), lambda i, j: (i, 0)),
        out_shape=jax.ShapeDtypeStruct((n, 1), jnp.float32),
        compiler_params=pltpu.CompilerParams(
            dimension_semantics=("parallel", "arbitrary")),
    )(score_col, score_row)


def _perm_body(rank_ref, perm_ref, *, bq, bi):
    r = pl.program_id(0)
    i = pl.program_id(1)
    rg = (r * bq + jax.lax.broadcasted_iota(jnp.int32, (bq, bi), 0)).astype(
        jnp.float32)
    ig = (i * bi + jax.lax.broadcasted_iota(jnp.int32, (bq, bi), 1)).astype(
        jnp.float32)
    eq = (rank_ref[...] == rg)
    contrib = jnp.sum(jnp.where(eq, ig, 0.0), axis=1, keepdims=True)

    @pl.when(i == 0)
    def _():
        perm_ref[...] = contrib

    @pl.when(i != 0)
    def _():
        perm_ref[...] += contrib


def _perm_from_rank(rank_row, k, bq=256, bi=512):
    n = rank_row.shape[1]
    bq = min(bq, k)
    bi = min(bi, n)
    return pl.pallas_call(
        functools.partial(_perm_body, bq=bq, bi=bi),
        grid=(k // bq, n // bi),
        in_specs=[pl.BlockSpec((1, bi), lambda r, i: (0, i))],
        out_specs=pl.BlockSpec((bq, 1), lambda r, i: (r, 0)),
        out_shape=jax.ShapeDtypeStruct((k, 1), jnp.float32),
        compiler_params=pltpu.CompilerParams(
            dimension_semantics=("parallel", "arbitrary")),
    )(rank_row)


def _gather_body(pref, src_ref, out_ref, *, binarize):
    if binarize:
        out_ref[...] = (src_ref[...] != 0.0).astype(jnp.float32)
    else:
        out_ref[...] = src_ref[...]


def _row_gather(src, perm, binarize=False):
    k = perm.shape[0]
    n, d = src.shape
    src3 = jnp.reshape(src, (n, 1, d))
    grid_spec = pltpu.PrefetchScalarGridSpec(
        num_scalar_prefetch=1,
        grid=(k,),
        in_specs=[pl.BlockSpec((1, 1, d), lambda i, pref: (pref[i], 0, 0))],
        out_specs=pl.BlockSpec((1, 1, d), lambda i, pref: (i, 0, 0)),
    )
    out = pl.pallas_call(
        functools.partial(_gather_body, binarize=binarize),
        grid_spec=grid_spec,
        out_shape=jax.ShapeDtypeStruct((k, 1, d), jnp.float32),
    )(perm, src3)
    return jnp.reshape(out, (k, d))


def _tr_body(src_ref, out_ref):
    out_ref[...] = src_ref[...].T


def _transpose(src, bm=256):
    m, n = src.shape
    bi = min(bm, n)
    bj = min(bm, m)
    return pl.pallas_call(
        _tr_body,
        grid=(n // bi, m // bj),
        in_specs=[pl.BlockSpec((bj, bi), lambda i, j: (j, i))],
        out_specs=pl.BlockSpec((bi, bj), lambda i, j: (i, j)),
        out_shape=jax.ShapeDtypeStruct((n, m), jnp.float32),
    )(src)


def _head_body(x_ref, w1_ref, b1_ref, w2_ref, b2_ref, w3_ref, b3_ref, o_ref):
    h = jnp.mean(x_ref[...], axis=0, keepdims=True)
    h = jnp.dot(h, w1_ref[...], preferred_element_type=jnp.float32) + b1_ref[...]
    h = jnp.maximum(h, 0.0)
    h = jnp.dot(h, w2_ref[...], preferred_element_type=jnp.float32) + b2_ref[...]
    h = jnp.maximum(h, 0.0)
    h = jnp.dot(h, w3_ref[...], preferred_element_type=jnp.float32) + b3_ref[...]
    o_ref[...] = h


def _head(x, w1, b1, w2, b2, w3, b3):
    n, d = x.shape
    return pl.pallas_call(
        _head_body,
        out_shape=jax.ShapeDtypeStruct((1, w3.shape[1]), jnp.float32),
    )(x, w1, b1, w2, b2, w3, b3)


def _pad_row(v, width=128):
    out = jnp.zeros((1, width), jnp.float32)
    return out.at[0, : v.shape[0]].set(v.astype(jnp.float32))


@jax.jit
def _run(x, edge_index, batch, params):
    n = x.shape[0]
    adj = jnp.zeros((n, n), jnp.float32).at[edge_index[1], edge_index[0]].set(1.0)

    for lp in params["layers"]:
        c = jnp.cumprod(lp["w"].astype(jnp.float32))
        c_pad = _pad_row(c)
        beta_pad = _pad_row(lp["beta"])
        wt = jnp.transpose(lp["W"]).astype(jnp.float32)
        b2d = lp["b"].astype(jnp.float32).reshape(1, -1)
        p2d = lp["p"].astype(jnp.float32).reshape(1, -1)

        b2 = _mm(adj, adj)
        b3 = _mm(b2, adj)
        b4 = _mm(b2, b2)
        total, nnz = _total_nnz(c_pad, adj, b2, b3, b4)

        dis = jnp.maximum(nnz, 1.0) ** -0.5
        m, s2 = _m_and_colsum(total, dis, dis.reshape(1, -1))

        relu_out, pre = _xside(m, x, wt, b2d, p2d,
                               jnp.reshape(s2, (-1, 1)), beta_pad)
        score = jnp.tanh(pre)
        xs_full = _scale_rows(relu_out, score)

        k = int(math.ceil(_RATIO * n))
        rank = _rank(score, jnp.reshape(score, (1, -1)))
        perm_f = _perm_from_rank(jnp.reshape(rank, (1, -1)), k)
        perm = perm_f.astype(jnp.int32).reshape(-1)

        x = _row_gather(xs_full, perm)
        if lp is not params["layers"][-1]:
            g = _row_gather(total, perm)
            h = _transpose(g)
            adj = _row_gather(h, perm, binarize=True)
        n = k

    fc = params["fc"]
    w1 = jnp.transpose(fc[0]["W"]).astype(jnp.float32)
    b1 = fc[0]["b"].reshape(1, -1)
    w2 = jnp.zeros((128, 128), jnp.float32).at[:, :64].set(
        jnp.transpose(fc[1]["W"]))
    b2w = jnp.zeros((1, 128), jnp.float32).at[0, :64].set(fc[1]["b"])
    w3 = jnp.zeros((128, 128), jnp.float32).at[:64, :10].set(
        jnp.transpose(fc[2]["W"]))
    b3w = jnp.zeros((1, 128), jnp.float32).at[0, :10].set(fc[2]["b"])
    out = _head(x, w1, b1, w2, b2w, w3, b3w)
    return out[:, :10]


def kernel(x, edge_index, batch, params):
    return _run(x, edge_index, batch, params)


# speedup vs baseline: 3.2941x; 3.2941x over previous
import functools
import math

import jax
import jax.numpy as jnp
from jax.experimental import pallas as pl
from jax.experimental.pallas import tpu as pltpu

_RATIO = 0.5
_FS = 4


def _cdiv(a, b):
    return (a + b - 1) // b


def _chain_body(cw_ref, left_ref, a_ref, out_ref, *, pre_idx, post_idx):
    left = left_ref[...]
    if pre_idx is not None:
        left = left * cw_ref[0, pre_idx]
    out_ref[...] = jnp.dot(left, a_ref[...],
                           preferred_element_type=jnp.float32) * cw_ref[0, post_idx]


def _chain_mm(cw_pad, left, a, pre_idx, post_idx, bm=512, bn=512):
    n = a.shape[0]
    bm = min(bm, n)
    bn = min(bn, n)
    return pl.pallas_call(
        functools.partial(_chain_body, pre_idx=pre_idx, post_idx=post_idx),
        grid=(n // bm, n // bn),
        in_specs=[
            pl.BlockSpec((1, 128), lambda i, j: (0, 0)),
            pl.BlockSpec((bm, n), lambda i, j: (i, 0)),
            pl.BlockSpec((n, bn), lambda i, j: (0, j)),
        ],
        out_specs=pl.BlockSpec((bm, bn), lambda i, j: (i, j)),
        out_shape=jax.ShapeDtypeStruct((n, n), jnp.float32),
        compiler_params=pltpu.CompilerParams(
            dimension_semantics=("parallel", "parallel")),
    )(cw_pad, left, a)


def _last_body(cw_ref, t3_ref, a_full_ref, aij_ref, t2_ref, t3ij_ref,
               tot_ref, nnz_ref):
    i = pl.program_id(0)
    j = pl.program_id(1)
    tmp4 = jnp.dot(t3_ref[...], a_full_ref[...],
                   preferred_element_type=jnp.float32) * cw_ref[0, 4]
    w0 = cw_ref[0, 0]
    c1 = cw_ref[0, 5]
    t = c1 * aij_ref[...]

    @pl.when(i == j)
    def _():
        rows = jax.lax.broadcasted_iota(jnp.int32, tmp4.shape, 0)
        cols = jax.lax.broadcasted_iota(jnp.int32, tmp4.shape, 1)
        tot_ref[...] = (((jnp.where(rows == cols, w0, 0.0) + t)
                         + t2_ref[...]) + t3ij_ref[...]) + tmp4

    @pl.when(i != j)
    def _():
        tot_ref[...] = ((t + t2_ref[...]) + t3ij_ref[...]) + tmp4

    cnt = jnp.sum((tot_ref[...] != 0.0).astype(jnp.float32), axis=1,
                  keepdims=True)

    @pl.when(j == 0)
    def _():
        nnz_ref[...] = cnt

    @pl.when(j != 0)
    def _():
        nnz_ref[...] += cnt


def _total_nnz(cw_pad, a, t2, t3, bm=512, bn=512):
    n = a.shape[0]
    bm = min(bm, n)
    bn = min(bn, n)
    ij = pl.BlockSpec((bm, bn), lambda i, j: (i, j))
    return pl.pallas_call(
        _last_body,
        grid=(n // bm, n // bn),
        in_specs=[
            pl.BlockSpec((1, 128), lambda i, j: (0, 0)),
            pl.BlockSpec((bm, n), lambda i, j: (i, 0)),
            pl.BlockSpec((n, bn), lambda i, j: (0, j)),
            ij, ij, ij,
        ],
        out_specs=[
            ij,
            pl.BlockSpec((bm, 1), lambda i, j: (i, 0)),
        ],
        out_shape=[
            jax.ShapeDtypeStruct((n, n), jnp.float32),
            jax.ShapeDtypeStruct((n, 1), jnp.float32),
        ],
        compiler_params=pltpu.CompilerParams(
            dimension_semantics=("parallel", "arbitrary")),
    )(cw_pad, t3, a, a, t2, t3)


def _m_body(tot_ref, dr_ref, dc_ref, m_ref, s2_ref):
    m = tot_ref[...] * dr_ref[...] * dc_ref[...]
    m_ref[...] = m
    s2_ref[...] = jnp.sum(m, axis=0, keepdims=True)


def _m_and_colsum(total, dis_r, dis_c, bn=512):
    n = total.shape[0]
    bn = min(bn, n)
    return pl.pallas_call(
        _m_body,
        grid=(n // bn,),
        in_specs=[
            pl.BlockSpec((n, bn), lambda j: (0, j)),
            pl.BlockSpec((n, 1), lambda j: (0, 0)),
            pl.BlockSpec((1, bn), lambda j: (0, j)),
        ],
        out_specs=[
            pl.BlockSpec((n, bn), lambda j: (0, j)),
            pl.BlockSpec((1, bn), lambda j: (0, j)),
        ],
        out_shape=[
            jax.ShapeDtypeStruct((n, n), jnp.float32),
            jax.ShapeDtypeStruct((1, n), jnp.float32),
        ],
        compiler_params=pltpu.CompilerParams(
            dimension_semantics=("parallel",)),
    )(total, dis_r, dis_c)


def _xside_body(m_ref, x_ref, wt_ref, b_ref, p_ref, s2_ref, beta_ref,
                xs_ref, sc_ref):
    mx = jnp.dot(m_ref[...], x_ref[...], preferred_element_type=jnp.float32)
    lin = jnp.dot(mx, wt_ref[...], preferred_element_type=jnp.float32) + b_ref[...]
    r = jnp.maximum(lin, 0.0)
    s1 = jnp.sum(r * p_ref[...], axis=1, keepdims=True)
    xs_ref[...] = r
    sc_ref[...] = beta_ref[0, 0] * s1 + beta_ref[0, 1] * s2_ref[...]


def _xside(m, x, wt, b2d, p2d, s2col, beta_pad, bm=512):
    n = m.shape[0]
    d_in = x.shape[1]
    d_out = wt.shape[1]
    bm = min(bm, n)
    xs, sc = pl.pallas_call(
        _xside_body,
        grid=(n // bm,),
        in_specs=[
            pl.BlockSpec((bm, n), lambda i: (i, 0)),
            pl.BlockSpec((n, d_in), lambda i: (0, 0)),
            pl.BlockSpec((d_in, d_out), lambda i: (0, 0)),
            pl.BlockSpec((1, d_out), lambda i: (0, 0)),
            pl.BlockSpec((1, d_out), lambda i: (0, 0)),
            pl.BlockSpec((bm, 1), lambda i: (i, 0)),
            pl.BlockSpec((1, 128), lambda i: (0, 0)),
        ],
        out_specs=[
            pl.BlockSpec((bm, d_out), lambda i: (i, 0)),
            pl.BlockSpec((bm, 1), lambda i: (i, 0)),
        ],
        out_shape=[
            jax.ShapeDtypeStruct((n, d_out), jnp.float32),
            jax.ShapeDtypeStruct((n, 1), jnp.float32),
        ],
        compiler_params=pltpu.CompilerParams(
            dimension_semantics=("parallel",)),
    )(m, x, wt, b2d, p2d, s2col, beta_pad)
    return xs, sc


def _scale_body(r_ref, sc_ref, o_ref):
    o_ref[...] = r_ref[...] * sc_ref[...]


def _scale_rows(r, score, bm=512):
    n, d = r.shape
    bm = min(bm, n)
    return pl.pallas_call(
        _scale_body,
        grid=(n // bm,),
        in_specs=[
            pl.BlockSpec((bm, d), lambda i: (i, 0)),
            pl.BlockSpec((bm, 1), lambda i: (i, 0)),
        ],
        out_specs=pl.BlockSpec((bm, d), lambda i: (i, 0)),
        out_shape=jax.ShapeDtypeStruct((n, d), jnp.float32),
    )(r, score)


def _rank_body(sr_ref, sc_ref, rank_ref, *, bm, bn):
    i = pl.program_id(0)
    j = pl.program_id(1)
    sr = sr_ref[...]
    sc = sc_ref[...]
    ig = i * bm + jax.lax.broadcasted_iota(jnp.int32, (bm, bn), 0)
    jg = j * bn + jax.lax.broadcasted_iota(jnp.int32, (bm, bn), 1)
    gt = (sc > sr)
    eq_lt = (sc == sr) & (jg < ig)
    contrib = jnp.sum((gt | eq_lt).astype(jnp.float32), axis=1, keepdims=True)

    @pl.when(j == 0)
    def _():
        rank_ref[...] = contrib

    @pl.when(j != 0)
    def _():
        rank_ref[...] += contrib


def _rank(score_col, score_row, bm=512):
    n = score_col.shape[0]
    bm = min(bm, n)
    return pl.pallas_call(
        functools.partial(_rank_body, bm=bm, bn=bm),
        grid=(n // bm, n // bm),
        in_specs=[
            pl.BlockSpec((bm, 1), lambda i, j: (i, 0)),
            pl.BlockSpec((1, bm), lambda i, j: (0, j)),
        ],
        out_specs=pl.BlockSpec((bm, 1), lambda i, j: (i, 0)),
        out_shape=jax.ShapeDtypeStruct((n, 1), jnp.float32),
        compiler_params=pltpu.CompilerParams(
            dimension_semantics=("parallel", "arbitrary")),
    )(score_col, score_row)


def _perm_body(rank_ref, perm_ref, *, bq, bi):
    r = pl.program_id(0)
    i = pl.program_id(1)
    rg = (r * bq + jax.lax.broadcasted_iota(jnp.int32, (bq, bi), 0)).astype(
        jnp.float32)
    ig = (i * bi + jax.lax.broadcasted_iota(jnp.int32, (bq, bi), 1)).astype(
        jnp.float32)
    eq = (rank_ref[...] == rg)
    contrib = jnp.sum(jnp.where(eq, ig, 0.0), axis=1, keepdims=True)

    @pl.when(i == 0)
    def _():
        perm_ref[...] = contrib

    @pl.when(i != 0)
    def _():
        perm_ref[...] += contrib


def _perm_from_rank(rank_row, k, bq=256, bi=512):
    n = rank_row.shape[1]
    bq = min(bq, k)
    bi = min(bi, n)
    return pl.pallas_call(
        functools.partial(_perm_body, bq=bq, bi=bi),
        grid=(k // bq, n // bi),
        in_specs=[pl.BlockSpec((1, bi), lambda r, i: (0, i))],
        out_specs=pl.BlockSpec((bq, 1), lambda r, i: (r, 0)),
        out_shape=jax.ShapeDtypeStruct((k, 1), jnp.float32),
        compiler_params=pltpu.CompilerParams(
            dimension_semantics=("parallel", "arbitrary")),
    )(rank_row)


_GR = 8


def _gather_body(pref, *refs):
    out_ref = refs[-1]
    for r in range(_GR):
        out_ref[r, 0, :] = refs[r][0, 0, :]


def _row_gather(src, perm):
    k = perm.shape[0]
    n, d = src.shape
    src3 = jnp.reshape(src, (n, 1, d))
    in_specs = [
        pl.BlockSpec((1, 1, d),
                     (lambda i, pref, r=r: (pref[i * _GR + r], 0, 0)))
        for r in range(_GR)
    ]
    grid_spec = pltpu.PrefetchScalarGridSpec(
        num_scalar_prefetch=1,
        grid=(k // _GR,),
        in_specs=in_specs,
        out_specs=pl.BlockSpec((_GR, 1, d), lambda i, pref: (i, 0, 0)),
    )
    out = pl.pallas_call(
        _gather_body,
        grid_spec=grid_spec,
        out_shape=jax.ShapeDtypeStruct((k, 1, d), jnp.float32),
    )(perm, *([src3] * _GR))
    return jnp.reshape(out, (k, d))


def _tr_body(src_ref, out_ref, *, binarize):
    t = src_ref[...].T
    if binarize:
        out_ref[...] = (t != 0.0).astype(jnp.float32)
    else:
        out_ref[...] = t


def _transpose(src, bm=256, binarize=False):
    m, n = src.shape
    bi = min(bm, n)
    bj = min(bm, m)
    return pl.pallas_call(
        functools.partial(_tr_body, binarize=binarize),
        grid=(n // bi, m // bj),
        in_specs=[pl.BlockSpec((bj, bi), lambda i, j: (j, i))],
        out_specs=pl.BlockSpec((bi, bj), lambda i, j: (i, j)),
        out_shape=jax.ShapeDtypeStruct((n, m), jnp.float32),
    )(src)


def _head_body(x_ref, w1_ref, b1_ref, w2_ref, b2_ref, w3_ref, b3_ref, o_ref):
    h = jnp.mean(x_ref[...], axis=0, keepdims=True)
    h = jnp.dot(h, w1_ref[...], preferred_element_type=jnp.float32) + b1_ref[...]
    h = jnp.maximum(h, 0.0)
    h = jnp.dot(h, w2_ref[...], preferred_element_type=jnp.float32) + b2_ref[...]
    h = jnp.maximum(h, 0.0)
    h = jnp.dot(h, w3_ref[...], preferred_element_type=jnp.float32) + b3_ref[...]
    o_ref[...] = h


def _head(x, w1, b1, w2, b2, w3, b3):
    n, d = x.shape
    return pl.pallas_call(
        _head_body,
        out_shape=jax.ShapeDtypeStruct((1, w3.shape[1]), jnp.float32),
    )(x, w1, b1, w2, b2, w3, b3)


def _pad_row(v, width=128):
    out = jnp.zeros((1, width), jnp.float32)
    return out.at[0, : v.shape[0]].set(v.astype(jnp.float32))


@jax.jit
def _run(x, edge_index, batch, params):
    n = x.shape[0]
    adj = jnp.zeros((n, n), jnp.float32).at[edge_index[1], edge_index[0]].set(1.0)

    for lp in params["layers"]:
        w = lp["w"].astype(jnp.float32)
        c1 = w[0] * w[1]
        cw_pad = _pad_row(jnp.concatenate([w, c1[None]]))
        beta_pad = _pad_row(lp["beta"])
        wt = jnp.transpose(lp["W"]).astype(jnp.float32)
        b2d = lp["b"].astype(jnp.float32).reshape(1, -1)
        p2d = lp["p"].astype(jnp.float32).reshape(1, -1)

        t2 = _chain_mm(cw_pad, adj, adj, pre_idx=5, post_idx=2)
        t3 = _chain_mm(cw_pad, t2, adj, pre_idx=None, post_idx=3)
        total, nnz = _total_nnz(cw_pad, adj, t2, t3)

        dis = jnp.maximum(nnz, 1.0) ** -0.5
        m, s2 = _m_and_colsum(total, dis, dis.reshape(1, -1))

        relu_out, pre = _xside(m, x, wt, b2d, p2d,
                               jnp.reshape(s2, (-1, 1)), beta_pad)
        score = jnp.tanh(pre)
        xs_full = _scale_rows(relu_out, score)

        k = int(math.ceil(_RATIO * n))
        rank = _rank(score, jnp.reshape(score, (1, -1)))
        perm_f = _perm_from_rank(jnp.reshape(rank, (1, -1)), k)
        perm = perm_f.astype(jnp.int32).reshape(-1)

        x = _row_gather(xs_full, perm)
        if lp is not params["layers"][-1]:
            g = _row_gather(total, perm)
            h = _transpose(g, binarize=True)
            adj = _row_gather(h, perm)
        n = k

    fc = params["fc"]
    w1 = jnp.transpose(fc[0]["W"]).astype(jnp.float32)
    b1 = fc[0]["b"].reshape(1, -1)
    w2 = jnp.zeros((128, 128), jnp.float32).at[:, :64].set(
        jnp.transpose(fc[1]["W"]))
    b2w = jnp.zeros((1, 128), jnp.float32).at[0, :64].set(fc[1]["b"])
    w3 = jnp.zeros((128, 128), jnp.float32).at[:64, :10].set(
        jnp.transpose(fc[2]["W"]))
    b3w = jnp.zeros((1, 128), jnp.float32).at[0, :10].set(fc[2]["b"])
    out = _head(x, w1, b1, w2, b2w, w3, b3w)
    return out[:, :10]


def kernel(x, edge_index, batch, params):
    return _run(x, edge_index, batch, params)


# speedup vs baseline: 3.4587x; 1.0500x over previous
import functools
import math

import jax
import jax.numpy as jnp
from jax.experimental import pallas as pl
from jax.experimental.pallas import tpu as pltpu

_RATIO = 0.5
_FS = 4


def _cdiv(a, b):
    return (a + b - 1) // b


def _chain_body(cw_ref, left_ref, a_ref, out_ref, *, pre_idx, post_idx):
    left = left_ref[...]
    if pre_idx is not None:
        left = left * cw_ref[0, pre_idx]
    out_ref[...] = jnp.dot(left, a_ref[...],
                           preferred_element_type=jnp.float32) * cw_ref[0, post_idx]


def _chain_mm(cw_pad, left, a, pre_idx, post_idx, bm=1024, bn=1024):
    n = a.shape[0]
    bm = min(bm, n)
    bn = min(bn, n)
    return pl.pallas_call(
        functools.partial(_chain_body, pre_idx=pre_idx, post_idx=post_idx),
        grid=(n // bm, n // bn),
        in_specs=[
            pl.BlockSpec((1, 128), lambda i, j: (0, 0)),
            pl.BlockSpec((bm, n), lambda i, j: (i, 0)),
            pl.BlockSpec((n, bn), lambda i, j: (0, j)),
        ],
        out_specs=pl.BlockSpec((bm, bn), lambda i, j: (i, j)),
        out_shape=jax.ShapeDtypeStruct((n, n), jnp.float32),
        compiler_params=pltpu.CompilerParams(
            dimension_semantics=("parallel", "parallel")),
    )(cw_pad, left, a)


def _last_body(cw_ref, t3_ref, a_full_ref, aij_ref, t2_ref, t3ij_ref,
               tot_ref, nnz_ref):
    i = pl.program_id(0)
    j = pl.program_id(1)
    tmp4 = jnp.dot(t3_ref[...], a_full_ref[...],
                   preferred_element_type=jnp.float32) * cw_ref[0, 4]
    w0 = cw_ref[0, 0]
    c1 = cw_ref[0, 5]
    t = c1 * aij_ref[...]

    @pl.when(i == j)
    def _():
        rows = jax.lax.broadcasted_iota(jnp.int32, tmp4.shape, 0)
        cols = jax.lax.broadcasted_iota(jnp.int32, tmp4.shape, 1)
        tot_ref[...] = (((jnp.where(rows == cols, w0, 0.0) + t)
                         + t2_ref[...]) + t3ij_ref[...]) + tmp4

    @pl.when(i != j)
    def _():
        tot_ref[...] = ((t + t2_ref[...]) + t3ij_ref[...]) + tmp4

    cnt = jnp.sum((tot_ref[...] != 0.0).astype(jnp.float32), axis=1,
                  keepdims=True)

    @pl.when(j == 0)
    def _():
        nnz_ref[...] = cnt

    @pl.when(j != 0)
    def _():
        nnz_ref[...] += cnt


def _total_nnz(cw_pad, a, t2, t3, bm=1024, bn=512):
    n = a.shape[0]
    bm = min(bm, n)
    bn = min(bn, n)
    ij = pl.BlockSpec((bm, bn), lambda i, j: (i, j))
    return pl.pallas_call(
        _last_body,
        grid=(n // bm, n // bn),
        in_specs=[
            pl.BlockSpec((1, 128), lambda i, j: (0, 0)),
            pl.BlockSpec((bm, n), lambda i, j: (i, 0)),
            pl.BlockSpec((n, bn), lambda i, j: (0, j)),
            ij, ij, ij,
        ],
        out_specs=[
            ij,
            pl.BlockSpec((bm, 1), lambda i, j: (i, 0)),
        ],
        out_shape=[
            jax.ShapeDtypeStruct((n, n), jnp.float32),
            jax.ShapeDtypeStruct((n, 1), jnp.float32),
        ],
        compiler_params=pltpu.CompilerParams(
            dimension_semantics=("parallel", "arbitrary")),
    )(cw_pad, t3, a, a, t2, t3)


def _m_body(tot_ref, dr_ref, dc_ref, m_ref, s2_ref):
    m = tot_ref[...] * dr_ref[...] * dc_ref[...]
    m_ref[...] = m
    s2_ref[...] = jnp.sum(m, axis=0, keepdims=True)


def _m_and_colsum(total, dis_r, dis_c, bn=512):
    n = total.shape[0]
    bn = min(bn, n)
    return pl.pallas_call(
        _m_body,
        grid=(n // bn,),
        in_specs=[
            pl.BlockSpec((n, bn), lambda j: (0, j)),
            pl.BlockSpec((n, 1), lambda j: (0, 0)),
            pl.BlockSpec((1, bn), lambda j: (0, j)),
        ],
        out_specs=[
            pl.BlockSpec((n, bn), lambda j: (0, j)),
            pl.BlockSpec((1, bn), lambda j: (0, j)),
        ],
        out_shape=[
            jax.ShapeDtypeStruct((n, n), jnp.float32),
            jax.ShapeDtypeStruct((1, n), jnp.float32),
        ],
        compiler_params=pltpu.CompilerParams(
            dimension_semantics=("parallel",)),
    )(total, dis_r, dis_c)


def _xside_body(m_ref, x_ref, wt_ref, b_ref, p_ref, s2_ref, beta_ref,
                xs_ref, sc_ref):
    mx = jnp.dot(m_ref[...], x_ref[...], preferred_element_type=jnp.float32)
    lin = jnp.dot(mx, wt_ref[...], preferred_element_type=jnp.float32) + b_ref[...]
    r = jnp.maximum(lin, 0.0)
    s1 = jnp.sum(r * p_ref[...], axis=1, keepdims=True)
    xs_ref[...] = r
    sc_ref[...] = beta_ref[0, 0] * s1 + beta_ref[0, 1] * s2_ref[...]


def _xside(m, x, wt, b2d, p2d, s2col, beta_pad, bm=1024):
    n = m.shape[0]
    d_in = x.shape[1]
    d_out = wt.shape[1]
    bm = min(bm, n)
    xs, sc = pl.pallas_call(
        _xside_body,
        grid=(n // bm,),
        in_specs=[
            pl.BlockSpec((bm, n), lambda i: (i, 0)),
            pl.BlockSpec((n, d_in), lambda i: (0, 0)),
            pl.BlockSpec((d_in, d_out), lambda i: (0, 0)),
            pl.BlockSpec((1, d_out), lambda i: (0, 0)),
            pl.BlockSpec((1, d_out), lambda i: (0, 0)),
            pl.BlockSpec((bm, 1), lambda i: (i, 0)),
            pl.BlockSpec((1, 128), lambda i: (0, 0)),
        ],
        out_specs=[
            pl.BlockSpec((bm, d_out), lambda i: (i, 0)),
            pl.BlockSpec((bm, 1), lambda i: (i, 0)),
        ],
        out_shape=[
            jax.ShapeDtypeStruct((n, d_out), jnp.float32),
            jax.ShapeDtypeStruct((n, 1), jnp.float32),
        ],
        compiler_params=pltpu.CompilerParams(
            dimension_semantics=("parallel",)),
    )(m, x, wt, b2d, p2d, s2col, beta_pad)
    return xs, sc


def _scale_body(r_ref, sc_ref, o_ref):
    o_ref[...] = r_ref[...] * sc_ref[...]


def _scale_rows(r, score, bm=512):
    n, d = r.shape
    bm = min(bm, n)
    return pl.pallas_call(
        _scale_body,
        grid=(n // bm,),
        in_specs=[
            pl.BlockSpec((bm, d), lambda i: (i, 0)),
            pl.BlockSpec((bm, 1), lambda i: (i, 0)),
        ],
        out_specs=pl.BlockSpec((bm, d), lambda i: (i, 0)),
        out_shape=jax.ShapeDtypeStruct((n, d), jnp.float32),
    )(r, score)


def _rank_body(sr_ref, sc_ref, rank_ref, *, bm, n):
    i = pl.program_id(0)
    sr = sr_ref[...]
    sc = sc_ref[...]
    ig = i * bm + jax.lax.broadcasted_iota(jnp.int32, (bm, n), 0)
    jg = jax.lax.broadcasted_iota(jnp.int32, (bm, n), 1)
    gt = (sc > sr)
    eq_lt = (sc == sr) & (jg < ig)
    rank_ref[...] = jnp.sum((gt | eq_lt).astype(jnp.float32), axis=1,
                            keepdims=True)


def _rank(score_col, score_row, bm=512):
    n = score_col.shape[0]
    bm = min(bm, n)
    return pl.pallas_call(
        functools.partial(_rank_body, bm=bm, n=n),
        grid=(n // bm,),
        in_specs=[
            pl.BlockSpec((bm, 1), lambda i: (i, 0)),
            pl.BlockSpec((1, n), lambda i: (0, 0)),
        ],
        out_specs=pl.BlockSpec((bm, 1), lambda i: (i, 0)),
        out_shape=jax.ShapeDtypeStruct((n, 1), jnp.float32),
        compiler_params=pltpu.CompilerParams(
            dimension_semantics=("parallel",)),
    )(score_col, score_row)


def _perm_body(rank_ref, perm_ref, *, bq, n):
    r = pl.program_id(0)
    rg = (r * bq + jax.lax.broadcasted_iota(jnp.int32, (bq, n), 0)).astype(
        jnp.float32)
    ig = jax.lax.broadcasted_iota(jnp.int32, (bq, n), 1).astype(jnp.float32)
    eq = (rank_ref[...] == rg)
    perm_ref[...] = jnp.sum(jnp.where(eq, ig, 0.0), axis=1, keepdims=True)


def _perm_from_rank(rank_row, k, bq=512):
    n = rank_row.shape[1]
    bq = min(bq, k)
    return pl.pallas_call(
        functools.partial(_perm_body, bq=bq, n=n),
        grid=(k // bq,),
        in_specs=[pl.BlockSpec((1, n), lambda r: (0, 0))],
        out_specs=pl.BlockSpec((bq, 1), lambda r: (r, 0)),
        out_shape=jax.ShapeDtypeStruct((k, 1), jnp.float32),
        compiler_params=pltpu.CompilerParams(
            dimension_semantics=("parallel",)),
    )(rank_row)


_GR = 8


def _gather_body(pref, *refs):
    out_ref = refs[-1]
    for r in range(_GR):
        out_ref[r, 0, :] = refs[r][0, 0, :]


def _row_gather(src, perm):
    k = perm.shape[0]
    n, d = src.shape
    src3 = jnp.reshape(src, (n, 1, d))
    in_specs = [
        pl.BlockSpec((1, 1, d),
                     (lambda i, pref, r=r: (pref[i * _GR + r], 0, 0)))
        for r in range(_GR)
    ]
    grid_spec = pltpu.PrefetchScalarGridSpec(
        num_scalar_prefetch=1,
        grid=(k // _GR,),
        in_specs=in_specs,
        out_specs=pl.BlockSpec((_GR, 1, d), lambda i, pref: (i, 0, 0)),
    )
    out = pl.pallas_call(
        _gather_body,
        grid_spec=grid_spec,
        out_shape=jax.ShapeDtypeStruct((k, 1, d), jnp.float32),
    )(perm, *([src3] * _GR))
    return jnp.reshape(out, (k, d))


def _tr_body(src_ref, out_ref, *, binarize):
    t = src_ref[...].T
    if binarize:
        out_ref[...] = (t != 0.0).astype(jnp.float32)
    else:
        out_ref[...] = t


def _transpose(src, bm=256, binarize=False):
    m, n = src.shape
    bi = min(bm, n)
    bj = min(bm, m)
    return pl.pallas_call(
        functools.partial(_tr_body, binarize=binarize),
        grid=(n // bi, m // bj),
        in_specs=[pl.BlockSpec((bj, bi), lambda i, j: (j, i))],
        out_specs=pl.BlockSpec((bi, bj), lambda i, j: (i, j)),
        out_shape=jax.ShapeDtypeStruct((n, m), jnp.float32),
    )(src)


def _head_body(x_ref, w1_ref, b1_ref, w2_ref, b2_ref, w3_ref, b3_ref, o_ref):
    h = jnp.mean(x_ref[...], axis=0, keepdims=True)
    h = jnp.dot(h, w1_ref[...], preferred_element_type=jnp.float32) + b1_ref[...]
    h = jnp.maximum(h, 0.0)
    h = jnp.dot(h, w2_ref[...], preferred_element_type=jnp.float32) + b2_ref[...]
    h = jnp.maximum(h, 0.0)
    h = jnp.dot(h, w3_ref[...], preferred_element_type=jnp.float32) + b3_ref[...]
    o_ref[...] = h


def _head(x, w1, b1, w2, b2, w3, b3):
    n, d = x.shape
    return pl.pallas_call(
        _head_body,
        out_shape=jax.ShapeDtypeStruct((1, w3.shape[1]), jnp.float32),
    )(x, w1, b1, w2, b2, w3, b3)


def _pad_row(v, width=128):
    out = jnp.zeros((1, width), jnp.float32)
    return out.at[0, : v.shape[0]].set(v.astype(jnp.float32))


@jax.jit
def _run(x, edge_index, batch, params):
    n = x.shape[0]
    adj = jnp.zeros((n, n), jnp.float32).at[edge_index[1], edge_index[0]].set(1.0)

    for lp in params["layers"]:
        w = lp["w"].astype(jnp.float32)
        c1 = w[0] * w[1]
        cw_pad = _pad_row(jnp.concatenate([w, c1[None]]))
        beta_pad = _pad_row(lp["beta"])
        wt = jnp.transpose(lp["W"]).astype(jnp.float32)
        b2d = lp["b"].astype(jnp.float32).reshape(1, -1)
        p2d = lp["p"].astype(jnp.float32).reshape(1, -1)

        t2 = _chain_mm(cw_pad, adj, adj, pre_idx=5, post_idx=2)
        t3 = _chain_mm(cw_pad, t2, adj, pre_idx=None, post_idx=3)
        total, nnz = _total_nnz(cw_pad, adj, t2, t3)

        dis = jnp.maximum(nnz, 1.0) ** -0.5
        m, s2 = _m_and_colsum(total, dis, dis.reshape(1, -1))

        relu_out, pre = _xside(m, x, wt, b2d, p2d,
                               jnp.reshape(s2, (-1, 1)), beta_pad)
        score = jnp.tanh(pre)
        xs_full = _scale_rows(relu_out, score)

        k = int(math.ceil(_RATIO * n))
        rank = _rank(score, jnp.reshape(score, (1, -1)))
        perm_f = _perm_from_rank(jnp.reshape(rank, (1, -1)), k)
        perm = perm_f.astype(jnp.int32).reshape(-1)

        x = _row_gather(xs_full, perm)
        if lp is not params["layers"][-1]:
            g = _row_gather(total, perm)
            h = _transpose(g, binarize=True)
            adj = _row_gather(h, perm)
        n = k

    fc = params["fc"]
    w1 = jnp.transpose(fc[0]["W"]).astype(jnp.float32)
    b1 = fc[0]["b"].reshape(1, -1)
    w2 = jnp.zeros((128, 128), jnp.float32).at[:, :64].set(
        jnp.transpose(fc[1]["W"]))
    b2w = jnp.zeros((1, 128), jnp.float32).at[0, :64].set(fc[1]["b"])
    w3 = jnp.zeros((128, 128), jnp.float32).at[:64, :10].set(
        jnp.transpose(fc[2]["W"]))
    b3w = jnp.zeros((1, 128), jnp.float32).at[0, :10].set(fc[2]["b"])
    out = _head(x, w1, b1, w2, b2w, w3, b3w)
    return out[:, :10]


def kernel(x, edge_index, batch, params):
    return _run(x, edge_index, batch, params)


# speedup vs baseline: 4.1140x; 1.1895x over previous
import functools
import math

import jax
import jax.numpy as jnp
from jax.experimental import pallas as pl
from jax.experimental.pallas import tpu as pltpu

_RATIO = 0.5
_FS = 4


def _cdiv(a, b):
    return (a + b - 1) // b


def _chain_body(cw_ref, left_ref, a_ref, out_ref, *, pre_idx, post_idx):
    left = left_ref[...]
    if pre_idx is not None:
        left = left * cw_ref[0, pre_idx]
    out_ref[...] = jnp.dot(left, a_ref[...],
                           preferred_element_type=jnp.float32) * cw_ref[0, post_idx]


def _chain_mm(cw_pad, left, a, pre_idx, post_idx, bm=1024, bn=1024):
    n = a.shape[0]
    bm = min(bm, n)
    bn = min(bn, n)
    return pl.pallas_call(
        functools.partial(_chain_body, pre_idx=pre_idx, post_idx=post_idx),
        grid=(n // bm, n // bn),
        in_specs=[
            pl.BlockSpec((1, 128), lambda i, j: (0, 0)),
            pl.BlockSpec((bm, n), lambda i, j: (i, 0)),
            pl.BlockSpec((n, bn), lambda i, j: (0, j)),
        ],
        out_specs=pl.BlockSpec((bm, bn), lambda i, j: (i, j)),
        out_shape=jax.ShapeDtypeStruct((n, n), jnp.float32),
        compiler_params=pltpu.CompilerParams(
            dimension_semantics=("parallel", "parallel")),
    )(cw_pad, left, a)


def _last_body(cw_ref, t3_ref, a_full_ref, aij_ref, t2_ref, t3ij_ref,
               tot_ref, nnz_ref):
    i = pl.program_id(0)
    j = pl.program_id(1)
    tmp4 = jnp.dot(t3_ref[...], a_full_ref[...],
                   preferred_element_type=jnp.float32) * cw_ref[0, 4]
    w0 = cw_ref[0, 0]
    c1 = cw_ref[0, 5]
    t = c1 * aij_ref[...]

    @pl.when(i == j)
    def _():
        rows = jax.lax.broadcasted_iota(jnp.int32, tmp4.shape, 0)
        cols = jax.lax.broadcasted_iota(jnp.int32, tmp4.shape, 1)
        tot_ref[...] = (((jnp.where(rows == cols, w0, 0.0) + t)
                         + t2_ref[...]) + t3ij_ref[...]) + tmp4

    @pl.when(i != j)
    def _():
        tot_ref[...] = ((t + t2_ref[...]) + t3ij_ref[...]) + tmp4

    cnt = jnp.sum((tot_ref[...] != 0.0).astype(jnp.float32), axis=1,
                  keepdims=True)

    @pl.when(j == 0)
    def _():
        nnz_ref[...] = cnt

    @pl.when(j != 0)
    def _():
        nnz_ref[...] += cnt


def _total_nnz(cw_pad, a, t2, t3, bm=1024, bn=512):
    n = a.shape[0]
    bm = min(bm, n)
    bn = min(bn, n)
    ij = pl.BlockSpec((bm, bn), lambda i, j: (i, j))
    return pl.pallas_call(
        _last_body,
        grid=(n // bm, n // bn),
        in_specs=[
            pl.BlockSpec((1, 128), lambda i, j: (0, 0)),
            pl.BlockSpec((bm, n), lambda i, j: (i, 0)),
            pl.BlockSpec((n, bn), lambda i, j: (0, j)),
            ij, ij, ij,
        ],
        out_specs=[
            ij,
            pl.BlockSpec((bm, 1), lambda i, j: (i, 0)),
        ],
        out_shape=[
            jax.ShapeDtypeStruct((n, n), jnp.float32),
            jax.ShapeDtypeStruct((n, 1), jnp.float32),
        ],
        compiler_params=pltpu.CompilerParams(
            dimension_semantics=("parallel", "arbitrary")),
    )(cw_pad, t3, a, a, t2, t3)


def _colsum_body(tot_ref, dr_ref, dc_ref, s2_ref):
    m = tot_ref[...] * dr_ref[...] * dc_ref[...]
    s2_ref[...] = jnp.sum(m, axis=0, keepdims=True)


def _m_colsum(total, dis_r, dis_c, bn=512):
    n = total.shape[0]
    bn = min(bn, n)
    return pl.pallas_call(
        _colsum_body,
        grid=(n // bn,),
        in_specs=[
            pl.BlockSpec((n, bn), lambda j: (0, j)),
            pl.BlockSpec((n, 1), lambda j: (0, 0)),
            pl.BlockSpec((1, bn), lambda j: (0, j)),
        ],
        out_specs=pl.BlockSpec((1, bn), lambda j: (0, j)),
        out_shape=jax.ShapeDtypeStruct((1, n), jnp.float32),
        compiler_params=pltpu.CompilerParams(
            dimension_semantics=("parallel",)),
    )(total, dis_r, dis_c)


def _xside_body(tot_ref, dr_ref, dc_ref, x_ref, wt_ref, b_ref, p_ref,
                s2_ref, beta_ref, xs_ref, sc_ref):
    m = tot_ref[...] * dr_ref[...] * dc_ref[...]
    mx = jnp.dot(m, x_ref[...], preferred_element_type=jnp.float32)
    lin = jnp.dot(mx, wt_ref[...], preferred_element_type=jnp.float32) + b_ref[...]
    r = jnp.maximum(lin, 0.0)
    s1 = jnp.sum(r * p_ref[...], axis=1, keepdims=True)
    xs_ref[...] = r
    sc_ref[...] = beta_ref[0, 0] * s1 + beta_ref[0, 1] * s2_ref[...]


def _xside(total, dis_r, dis_c, x, wt, b2d, p2d, s2col, beta_pad, bm=1024):
    n = total.shape[0]
    d_in = x.shape[1]
    d_out = wt.shape[1]
    bm = min(bm, n)
    xs, sc = pl.pallas_call(
        _xside_body,
        grid=(n // bm,),
        in_specs=[
            pl.BlockSpec((bm, n), lambda i: (i, 0)),
            pl.BlockSpec((bm, 1), lambda i: (i, 0)),
            pl.BlockSpec((1, n), lambda i: (0, 0)),
            pl.BlockSpec((n, d_in), lambda i: (0, 0)),
            pl.BlockSpec((d_in, d_out), lambda i: (0, 0)),
            pl.BlockSpec((1, d_out), lambda i: (0, 0)),
            pl.BlockSpec((1, d_out), lambda i: (0, 0)),
            pl.BlockSpec((bm, 1), lambda i: (i, 0)),
            pl.BlockSpec((1, 128), lambda i: (0, 0)),
        ],
        out_specs=[
            pl.BlockSpec((bm, d_out), lambda i: (i, 0)),
            pl.BlockSpec((bm, 1), lambda i: (i, 0)),
        ],
        out_shape=[
            jax.ShapeDtypeStruct((n, d_out), jnp.float32),
            jax.ShapeDtypeStruct((n, 1), jnp.float32),
        ],
        compiler_params=pltpu.CompilerParams(
            dimension_semantics=("parallel",)),
    )(total, dis_r, dis_c, x, wt, b2d, p2d, s2col, beta_pad)
    return xs, sc


def _scale_body(r_ref, sc_ref, o_ref):
    o_ref[...] = r_ref[...] * sc_ref[...]


def _scale_rows(r, score, bm=512):
    n, d = r.shape
    bm = min(bm, n)
    return pl.pallas_call(
        _scale_body,
        grid=(n // bm,),
        in_specs=[
            pl.BlockSpec((bm, d), lambda i: (i, 0)),
            pl.BlockSpec((bm, 1), lambda i: (i, 0)),
        ],
        out_specs=pl.BlockSpec((bm, d), lambda i: (i, 0)),
        out_shape=jax.ShapeDtypeStruct((n, d), jnp.float32),
    )(r, score)


def _rank_body(sr_ref, sc_ref, rank_ref, *, bm, n):
    i = pl.program_id(0)
    sr = sr_ref[...]
    sc = sc_ref[...]
    ig = i * bm + jax.lax.broadcasted_iota(jnp.int32, (bm, n), 0)
    jg = jax.lax.broadcasted_iota(jnp.int32, (bm, n), 1)
    gt = (sc > sr)
    eq_lt = (sc == sr) & (jg < ig)
    rank_ref[...] = jnp.sum((gt | eq_lt).astype(jnp.float32), axis=1,
                            keepdims=True)


def _rank(score_col, score_row, bm=512):
    n = score_col.shape[0]
    bm = min(bm, n)
    return pl.pallas_call(
        functools.partial(_rank_body, bm=bm, n=n),
        grid=(n // bm,),
        in_specs=[
            pl.BlockSpec((bm, 1), lambda i: (i, 0)),
            pl.BlockSpec((1, n), lambda i: (0, 0)),
        ],
        out_specs=pl.BlockSpec((bm, 1), lambda i: (i, 0)),
        out_shape=jax.ShapeDtypeStruct((n, 1), jnp.float32),
        compiler_params=pltpu.CompilerParams(
            dimension_semantics=("parallel",)),
    )(score_col, score_row)


def _perm_body(rank_ref, perm_ref, *, bq, n):
    r = pl.program_id(0)
    rg = (r * bq + jax.lax.broadcasted_iota(jnp.int32, (bq, n), 0)).astype(
        jnp.float32)
    ig = jax.lax.broadcasted_iota(jnp.int32, (bq, n), 1).astype(jnp.float32)
    eq = (rank_ref[...] == rg)
    perm_ref[...] = jnp.sum(jnp.where(eq, ig, 0.0), axis=1, keepdims=True)


def _perm_from_rank(rank_row, k, bq=512):
    n = rank_row.shape[1]
    bq = min(bq, k)
    return pl.pallas_call(
        functools.partial(_perm_body, bq=bq, n=n),
        grid=(k // bq,),
        in_specs=[pl.BlockSpec((1, n), lambda r: (0, 0))],
        out_specs=pl.BlockSpec((bq, 1), lambda r: (r, 0)),
        out_shape=jax.ShapeDtypeStruct((k, 1), jnp.float32),
        compiler_params=pltpu.CompilerParams(
            dimension_semantics=("parallel",)),
    )(rank_row)


_GR = 16


def _gather_body(pref, *refs):
    out_ref = refs[-1]
    for r in range(_GR):
        out_ref[r, 0, :] = refs[r][0, 0, :]


def _row_gather(src, perm):
    k = perm.shape[0]
    n, d = src.shape
    src3 = jnp.reshape(src, (n, 1, d))
    in_specs = [
        pl.BlockSpec((1, 1, d),
                     (lambda i, pref, r=r: (pref[i * _GR + r], 0, 0)))
        for r in range(_GR)
    ]
    grid_spec = pltpu.PrefetchScalarGridSpec(
        num_scalar_prefetch=1,
        grid=(k // _GR,),
        in_specs=in_specs,
        out_specs=pl.BlockSpec((_GR, 1, d), lambda i, pref: (i, 0, 0)),
    )
    out = pl.pallas_call(
        _gather_body,
        grid_spec=grid_spec,
        out_shape=jax.ShapeDtypeStruct((k, 1, d), jnp.float32),
    )(perm, *([src3] * _GR))
    return jnp.reshape(out, (k, d))


def _tr_body(src_ref, out_ref, *, binarize):
    t = src_ref[...].T
    if binarize:
        out_ref[...] = (t != 0.0).astype(jnp.float32)
    else:
        out_ref[...] = t


def _transpose(src, bm=256, binarize=False):
    m, n = src.shape
    bi = min(bm, n)
    bj = min(bm, m)
    return pl.pallas_call(
        functools.partial(_tr_body, binarize=binarize),
        grid=(n // bi, m // bj),
        in_specs=[pl.BlockSpec((bj, bi), lambda i, j: (j, i))],
        out_specs=pl.BlockSpec((bi, bj), lambda i, j: (i, j)),
        out_shape=jax.ShapeDtypeStruct((n, m), jnp.float32),
    )(src)


def _head_body(x_ref, w1_ref, b1_ref, w2_ref, b2_ref, w3_ref, b3_ref, o_ref):
    h = jnp.mean(x_ref[...], axis=0, keepdims=True)
    h = jnp.dot(h, w1_ref[...], preferred_element_type=jnp.float32) + b1_ref[...]
    h = jnp.maximum(h, 0.0)
    h = jnp.dot(h, w2_ref[...], preferred_element_type=jnp.float32) + b2_ref[...]
    h = jnp.maximum(h, 0.0)
    h = jnp.dot(h, w3_ref[...], preferred_element_type=jnp.float32) + b3_ref[...]
    o_ref[...] = h


def _head(x, w1, b1, w2, b2, w3, b3):
    n, d = x.shape
    return pl.pallas_call(
        _head_body,
        out_shape=jax.ShapeDtypeStruct((1, w3.shape[1]), jnp.float32),
    )(x, w1, b1, w2, b2, w3, b3)


def _pad_row(v, width=128):
    out = jnp.zeros((1, width), jnp.float32)
    return out.at[0, : v.shape[0]].set(v.astype(jnp.float32))


@jax.jit
def _run(x, edge_index, batch, params):
    n = x.shape[0]
    adj = jnp.zeros((n, n), jnp.float32).at[edge_index[1], edge_index[0]].set(1.0)

    for lp in params["layers"]:
        w = lp["w"].astype(jnp.float32)
        c1 = w[0] * w[1]
        cw_pad = _pad_row(jnp.concatenate([w, c1[None]]))
        beta_pad = _pad_row(lp["beta"])
        wt = jnp.transpose(lp["W"]).astype(jnp.float32)
        b2d = lp["b"].astype(jnp.float32).reshape(1, -1)
        p2d = lp["p"].astype(jnp.float32).reshape(1, -1)

        t2 = _chain_mm(cw_pad, adj, adj, pre_idx=5, post_idx=2)
        t3 = _chain_mm(cw_pad, t2, adj, pre_idx=None, post_idx=3)
        total, nnz = _total_nnz(cw_pad, adj, t2, t3)

        dis = jnp.maximum(nnz, 1.0) ** -0.5
        dis_c = dis.reshape(1, -1)
        s2 = _m_colsum(total, dis, dis_c)

        relu_out, pre = _xside(total, dis, dis_c, x, wt, b2d, p2d,
                               jnp.reshape(s2, (-1, 1)), beta_pad)
        score = jnp.tanh(pre)
        xs_full = _scale_rows(relu_out, score)

        k = int(math.ceil(_RATIO * n))
        rank = _rank(score, jnp.reshape(score, (1, -1)))
        perm_f = _perm_from_rank(jnp.reshape(rank, (1, -1)), k)
        perm = perm_f.astype(jnp.int32).reshape(-1)

        x = _row_gather(xs_full, perm)
        if lp is not params["layers"][-1]:
            g = _row_gather(total, perm)
            h = _transpose(g, binarize=True)
            adj = _row_gather(h, perm)
        n = k

    fc = params["fc"]
    w1 = jnp.transpose(fc[0]["W"]).astype(jnp.float32)
    b1 = fc[0]["b"].reshape(1, -1)
    w2 = jnp.zeros((128, 128), jnp.float32).at[:, :64].set(
        jnp.transpose(fc[1]["W"]))
    b2w = jnp.zeros((1, 128), jnp.float32).at[0, :64].set(fc[1]["b"])
    w3 = jnp.zeros((128, 128), jnp.float32).at[:64, :10].set(
        jnp.transpose(fc[2]["W"]))
    b3w = jnp.zeros((1, 128), jnp.float32).at[0, :10].set(fc[2]["b"])
    out = _head(x, w1, b1, w2, b2w, w3, b3w)
    return out[:, :10]


def kernel(x, edge_index, batch, params):
    return _run(x, edge_index, batch, params)


# speedup vs baseline: 4.5955x; 1.1170x over previous
import functools
import math

import jax
import jax.numpy as jnp
from jax.experimental import pallas as pl
from jax.experimental.pallas import tpu as pltpu

_RATIO = 0.5
_FS = 4


def _cdiv(a, b):
    return (a + b - 1) // b


def _chain_body(cw_ref, left_ref, a_ref, out_ref, *, pre_idx, post_idx):
    left = left_ref[...]
    if pre_idx is not None:
        left = left * cw_ref[0, pre_idx]
    out_ref[...] = jnp.dot(left, a_ref[...],
                           preferred_element_type=jnp.float32) * cw_ref[0, post_idx]


def _chain_mm(cw_pad, left, a, pre_idx, post_idx, bm=1024, bn=1024):
    n = a.shape[0]
    bm = min(bm, n)
    bn = min(bn, n)
    return pl.pallas_call(
        functools.partial(_chain_body, pre_idx=pre_idx, post_idx=post_idx),
        grid=(n // bm, n // bn),
        in_specs=[
            pl.BlockSpec((1, 128), lambda i, j: (0, 0)),
            pl.BlockSpec((bm, n), lambda i, j: (i, 0)),
            pl.BlockSpec((n, bn), lambda i, j: (0, j)),
        ],
        out_specs=pl.BlockSpec((bm, bn), lambda i, j: (i, j)),
        out_shape=jax.ShapeDtypeStruct((n, n), jnp.float32),
        compiler_params=pltpu.CompilerParams(
            dimension_semantics=("parallel", "parallel")),
    )(cw_pad, left, a)


def _last_body(cw_ref, t3_ref, a_full_ref, aij_ref, t2_ref, t3ij_ref,
               tot_ref, nnz_ref):
    i = pl.program_id(0)
    j = pl.program_id(1)
    tmp4 = jnp.dot(t3_ref[...], a_full_ref[...],
                   preferred_element_type=jnp.float32) * cw_ref[0, 4]
    w0 = cw_ref[0, 0]
    c1 = cw_ref[0, 5]
    t = c1 * aij_ref[...]

    @pl.when(i == j)
    def _():
        rows = jax.lax.broadcasted_iota(jnp.int32, tmp4.shape, 0)
        cols = jax.lax.broadcasted_iota(jnp.int32, tmp4.shape, 1)
        tot_ref[...] = (((jnp.where(rows == cols, w0, 0.0) + t)
                         + t2_ref[...]) + t3ij_ref[...]) + tmp4

    @pl.when(i != j)
    def _():
        tot_ref[...] = ((t + t2_ref[...]) + t3ij_ref[...]) + tmp4

    cnt = jnp.sum((tot_ref[...] != 0.0).astype(jnp.float32), axis=1,
                  keepdims=True)

    @pl.when(j == 0)
    def _():
        nnz_ref[...] = cnt

    @pl.when(j != 0)
    def _():
        nnz_ref[...] += cnt


def _total_nnz(cw_pad, a, t2, t3, bm=1024, bn=512):
    n = a.shape[0]
    bm = min(bm, n)
    bn = min(bn, n)
    ij = pl.BlockSpec((bm, bn), lambda i, j: (i, j))
    return pl.pallas_call(
        _last_body,
        grid=(n // bm, n // bn),
        in_specs=[
            pl.BlockSpec((1, 128), lambda i, j: (0, 0)),
            pl.BlockSpec((bm, n), lambda i, j: (i, 0)),
            pl.BlockSpec((n, bn), lambda i, j: (0, j)),
            ij, ij, ij,
        ],
        out_specs=[
            ij,
            pl.BlockSpec((bm, 1), lambda i, j: (i, 0)),
        ],
        out_shape=[
            jax.ShapeDtypeStruct((n, n), jnp.float32),
            jax.ShapeDtypeStruct((n, 1), jnp.float32),
        ],
        compiler_params=pltpu.CompilerParams(
            dimension_semantics=("parallel", "arbitrary")),
    )(cw_pad, t3, a, a, t2, t3)


def _colsum_body(tot_ref, dr_ref, dc_ref, s2_ref):
    m = tot_ref[...] * dr_ref[...] * dc_ref[...]
    s2_ref[...] = jnp.sum(m, axis=0, keepdims=True)


def _m_colsum(total, dis_r, dis_c, bn=512):
    n = total.shape[0]
    bn = min(bn, n)
    return pl.pallas_call(
        _colsum_body,
        grid=(n // bn,),
        in_specs=[
            pl.BlockSpec((n, bn), lambda j: (0, j)),
            pl.BlockSpec((n, 1), lambda j: (0, 0)),
            pl.BlockSpec((1, bn), lambda j: (0, j)),
        ],
        out_specs=pl.BlockSpec((1, bn), lambda j: (0, j)),
        out_shape=jax.ShapeDtypeStruct((1, n), jnp.float32),
        compiler_params=pltpu.CompilerParams(
            dimension_semantics=("parallel",)),
    )(total, dis_r, dis_c)


def _xside_body(tot_ref, dr_ref, dc_ref, x_ref, wt_ref, b_ref, p_ref,
                s2_ref, beta_ref, xs_ref, sc_ref):
    m = tot_ref[...] * dr_ref[...] * dc_ref[...]
    mx = jnp.dot(m, x_ref[...], preferred_element_type=jnp.float32)
    lin = jnp.dot(mx, wt_ref[...], preferred_element_type=jnp.float32) + b_ref[...]
    r = jnp.maximum(lin, 0.0)
    s1 = jnp.sum(r * p_ref[...], axis=1, keepdims=True)
    xs_ref[...] = r
    sc_ref[...] = beta_ref[0, 0] * s1 + beta_ref[0, 1] * s2_ref[...]


def _xside(total, dis_r, dis_c, x, wt, b2d, p2d, s2col, beta_pad, bm=1024):
    n = total.shape[0]
    d_in = x.shape[1]
    d_out = wt.shape[1]
    bm = min(bm, n)
    xs, sc = pl.pallas_call(
        _xside_body,
        grid=(n // bm,),
        in_specs=[
            pl.BlockSpec((bm, n), lambda i: (i, 0)),
            pl.BlockSpec((bm, 1), lambda i: (i, 0)),
            pl.BlockSpec((1, n), lambda i: (0, 0)),
            pl.BlockSpec((n, d_in), lambda i: (0, 0)),
            pl.BlockSpec((d_in, d_out), lambda i: (0, 0)),
            pl.BlockSpec((1, d_out), lambda i: (0, 0)),
            pl.BlockSpec((1, d_out), lambda i: (0, 0)),
            pl.BlockSpec((bm, 1), lambda i: (i, 0)),
            pl.BlockSpec((1, 128), lambda i: (0, 0)),
        ],
        out_specs=[
            pl.BlockSpec((bm, d_out), lambda i: (i, 0)),
            pl.BlockSpec((bm, 1), lambda i: (i, 0)),
        ],
        out_shape=[
            jax.ShapeDtypeStruct((n, d_out), jnp.float32),
            jax.ShapeDtypeStruct((n, 1), jnp.float32),
        ],
        compiler_params=pltpu.CompilerParams(
            dimension_semantics=("parallel",)),
    )(total, dis_r, dis_c, x, wt, b2d, p2d, s2col, beta_pad)
    return xs, sc


def _rank_body(sr_ref, sc_ref, r_ref, rank_ref, xs_ref, *, bm, n):
    i = pl.program_id(0)
    sr = sr_ref[...]
    sc = sc_ref[...]
    ig = i * bm + jax.lax.broadcasted_iota(jnp.int32, (bm, n), 0)
    jg = jax.lax.broadcasted_iota(jnp.int32, (bm, n), 1)
    gt = (sc > sr)
    eq_lt = (sc == sr) & (jg < ig)
    rank_ref[...] = jnp.sum((gt | eq_lt).astype(jnp.float32), axis=1,
                            keepdims=True)
    xs_ref[...] = r_ref[...] * sr


def _rank_scale(score_col, score_row, relu_out, bm=512):
    n = score_col.shape[0]
    d = relu_out.shape[1]
    bm = min(bm, n)
    return pl.pallas_call(
        functools.partial(_rank_body, bm=bm, n=n),
        grid=(n // bm,),
        in_specs=[
            pl.BlockSpec((bm, 1), lambda i: (i, 0)),
            pl.BlockSpec((1, n), lambda i: (0, 0)),
            pl.BlockSpec((bm, d), lambda i: (i, 0)),
        ],
        out_specs=[
            pl.BlockSpec((bm, 1), lambda i: (i, 0)),
            pl.BlockSpec((bm, d), lambda i: (i, 0)),
        ],
        out_shape=[
            jax.ShapeDtypeStruct((n, 1), jnp.float32),
            jax.ShapeDtypeStruct((n, d), jnp.float32),
        ],
        compiler_params=pltpu.CompilerParams(
            dimension_semantics=("parallel",)),
    )(score_col, score_row, relu_out)


def _perm_body(rank_ref, perm_ref, *, bq, n):
    r = pl.program_id(0)
    rg = (r * bq + jax.lax.broadcasted_iota(jnp.int32, (bq, n), 0)).astype(
        jnp.float32)
    ig = jax.lax.broadcasted_iota(jnp.int32, (bq, n), 1).astype(jnp.float32)
    eq = (rank_ref[...] == rg)
    perm_ref[...] = jnp.sum(jnp.where(eq, ig, 0.0), axis=1, keepdims=True)


def _perm_from_rank(rank_row, k, bq=512):
    n = rank_row.shape[1]
    bq = min(bq, k)
    return pl.pallas_call(
        functools.partial(_perm_body, bq=bq, n=n),
        grid=(k // bq,),
        in_specs=[pl.BlockSpec((1, n), lambda r: (0, 0))],
        out_specs=pl.BlockSpec((bq, 1), lambda r: (r, 0)),
        out_shape=jax.ShapeDtypeStruct((k, 1), jnp.float32),
        compiler_params=pltpu.CompilerParams(
            dimension_semantics=("parallel",)),
    )(rank_row)


_GR = 32


def _gather_body(pref, *refs):
    out_ref = refs[-1]
    for r in range(_GR):
        out_ref[r, 0, :] = refs[r][0, 0, :]


def _row_gather(src, perm):
    k = perm.shape[0]
    n, d = src.shape
    src3 = jnp.reshape(src, (n, 1, d))
    in_specs = [
        pl.BlockSpec((1, 1, d),
                     (lambda i, pref, r=r: (pref[i * _GR + r], 0, 0)))
        for r in range(_GR)
    ]
    grid_spec = pltpu.PrefetchScalarGridSpec(
        num_scalar_prefetch=1,
        grid=(k // _GR,),
        in_specs=in_specs,
        out_specs=pl.BlockSpec((_GR, 1, d), lambda i, pref: (i, 0, 0)),
    )
    out = pl.pallas_call(
        _gather_body,
        grid_spec=grid_spec,
        out_shape=jax.ShapeDtypeStruct((k, 1, d), jnp.float32),
    )(perm, *([src3] * _GR))
    return jnp.reshape(out, (k, d))


def _tr_body(src_ref, out_ref, *, binarize):
    t = src_ref[...].T
    if binarize:
        out_ref[...] = (t != 0.0).astype(jnp.float32)
    else:
        out_ref[...] = t


def _transpose(src, bm=256, binarize=False):
    m, n = src.shape
    bi = min(bm, n)
    bj = min(bm, m)
    return pl.pallas_call(
        functools.partial(_tr_body, binarize=binarize),
        grid=(n // bi, m // bj),
        in_specs=[pl.BlockSpec((bj, bi), lambda i, j: (j, i))],
        out_specs=pl.BlockSpec((bi, bj), lambda i, j: (i, j)),
        out_shape=jax.ShapeDtypeStruct((n, m), jnp.float32),
    )(src)


def _head_body(x_ref, w1_ref, b1_ref, w2_ref, b2_ref, w3_ref, b3_ref, o_ref):
    h = jnp.mean(x_ref[...], axis=0, keepdims=True)
    h = jnp.dot(h, w1_ref[...], preferred_element_type=jnp.float32) + b1_ref[...]
    h = jnp.maximum(h, 0.0)
    h = jnp.dot(h, w2_ref[...], preferred_element_type=jnp.float32) + b2_ref[...]
    h = jnp.maximum(h, 0.0)
    h = jnp.dot(h, w3_ref[...], preferred_element_type=jnp.float32) + b3_ref[...]
    o_ref[...] = h


def _head(x, w1, b1, w2, b2, w3, b3):
    n, d = x.shape
    return pl.pallas_call(
        _head_body,
        out_shape=jax.ShapeDtypeStruct((1, w3.shape[1]), jnp.float32),
    )(x, w1, b1, w2, b2, w3, b3)


def _pad_row(v, width=128):
    out = jnp.zeros((1, width), jnp.float32)
    return out.at[0, : v.shape[0]].set(v.astype(jnp.float32))


@jax.jit
def _run(x, edge_index, batch, params):
    n = x.shape[0]
    adj = jnp.zeros((n, n), jnp.float32).at[edge_index[1], edge_index[0]].set(1.0)

    for lp in params["layers"]:
        w = lp["w"].astype(jnp.float32)
        c1 = w[0] * w[1]
        cw_pad = _pad_row(jnp.concatenate([w, c1[None]]))
        beta_pad = _pad_row(lp["beta"])
        wt = jnp.transpose(lp["W"]).astype(jnp.float32)
        b2d = lp["b"].astype(jnp.float32).reshape(1, -1)
        p2d = lp["p"].astype(jnp.float32).reshape(1, -1)

        t2 = _chain_mm(cw_pad, adj, adj, pre_idx=5, post_idx=2)
        t3 = _chain_mm(cw_pad, t2, adj, pre_idx=None, post_idx=3)
        total, nnz = _total_nnz(cw_pad, adj, t2, t3)

        dis = jnp.maximum(nnz, 1.0) ** -0.5
        dis_c = dis.reshape(1, -1)
        s2 = _m_colsum(total, dis, dis_c)

        relu_out, pre = _xside(total, dis, dis_c, x, wt, b2d, p2d,
                               jnp.reshape(s2, (-1, 1)), beta_pad)
        score = jnp.tanh(pre)

        k = int(math.ceil(_RATIO * n))
        rank, xs_full = _rank_scale(score, jnp.reshape(score, (1, -1)),
                                    relu_out)
        perm_f = _perm_from_rank(jnp.reshape(rank, (1, -1)), k)
        perm = perm_f.astype(jnp.int32).reshape(-1)

        x = _row_gather(xs_full, perm)
        if lp is not params["layers"][-1]:
            g = _row_gather(total, perm)
            h = _transpose(g, binarize=True)
            adj = _row_gather(h, perm)
        n = k

    fc = params["fc"]
    w1 = jnp.transpose(fc[0]["W"]).astype(jnp.float32)
    b1 = fc[0]["b"].reshape(1, -1)
    w2 = jnp.zeros((128, 128), jnp.float32).at[:, :64].set(
        jnp.transpose(fc[1]["W"]))
    b2w = jnp.zeros((1, 128), jnp.float32).at[0, :64].set(fc[1]["b"])
    w3 = jnp.zeros((128, 128), jnp.float32).at[:64, :10].set(
        jnp.transpose(fc[2]["W"]))
    b3w = jnp.zeros((1, 128), jnp.float32).at[0, :10].set(fc[2]["b"])
    out = _head(x, w1, b1, w2, b2w, w3, b3w)
    return out[:, :10]


def kernel(x, edge_index, batch, params):
    return _run(x, edge_index, batch, params)


# speedup vs baseline: 4.6353x; 1.0087x over previous
import functools
import math

import jax
import jax.numpy as jnp
from jax.experimental import pallas as pl
from jax.experimental.pallas import tpu as pltpu

_RATIO = 0.5
_FS = 4


def _cdiv(a, b):
    return (a + b - 1) // b


def _chain_body(cw_ref, left_ref, a_ref, out_ref, *, pre_idx, post_idx):
    left = left_ref[...]
    if pre_idx is not None:
        left = left * cw_ref[0, pre_idx]
    out_ref[...] = jnp.dot(left, a_ref[...],
                           preferred_element_type=jnp.float32) * cw_ref[0, post_idx]


def _chain_mm(cw_pad, left, a, pre_idx, post_idx, bm=1024, bn=1024):
    n = a.shape[0]
    bm = min(bm, n)
    bn = min(bn, n)
    return pl.pallas_call(
        functools.partial(_chain_body, pre_idx=pre_idx, post_idx=post_idx),
        grid=(n // bm, n // bn),
        in_specs=[
            pl.BlockSpec((1, 128), lambda i, j: (0, 0)),
            pl.BlockSpec((bm, n), lambda i, j: (i, 0)),
            pl.BlockSpec((n, bn), lambda i, j: (0, j)),
        ],
        out_specs=pl.BlockSpec((bm, bn), lambda i, j: (i, j)),
        out_shape=jax.ShapeDtypeStruct((n, n), jnp.float32),
        compiler_params=pltpu.CompilerParams(
            dimension_semantics=("parallel", "parallel")),
    )(cw_pad, left, a)


def _last_body(cw_ref, t3_ref, a_full_ref, aij_ref, t2_ref, t3ij_ref,
               tot_ref, nnz_ref):
    i = pl.program_id(0)
    j = pl.program_id(1)
    tmp4 = jnp.dot(t3_ref[...], a_full_ref[...],
                   preferred_element_type=jnp.float32) * cw_ref[0, 4]
    w0 = cw_ref[0, 0]
    c1 = cw_ref[0, 5]
    t = c1 * aij_ref[...]

    @pl.when(i == j)
    def _():
        rows = jax.lax.broadcasted_iota(jnp.int32, tmp4.shape, 0)
        cols = jax.lax.broadcasted_iota(jnp.int32, tmp4.shape, 1)
        tot_ref[...] = (((jnp.where(rows == cols, w0, 0.0) + t)
                         + t2_ref[...]) + t3ij_ref[...]) + tmp4

    @pl.when(i != j)
    def _():
        tot_ref[...] = ((t + t2_ref[...]) + t3ij_ref[...]) + tmp4

    cnt = jnp.sum((tot_ref[...] != 0.0).astype(jnp.float32), axis=1,
                  keepdims=True)

    @pl.when(j == 0)
    def _():
        nnz_ref[...] = cnt

    @pl.when(j != 0)
    def _():
        nnz_ref[...] += cnt


def _total_nnz(cw_pad, a, t2, t3, bm=1024, bn=512):
    n = a.shape[0]
    bm = min(bm, n)
    bn = min(bn, n)
    ij = pl.BlockSpec((bm, bn), lambda i, j: (i, j))
    return pl.pallas_call(
        _last_body,
        grid=(n // bm, n // bn),
        in_specs=[
            pl.BlockSpec((1, 128), lambda i, j: (0, 0)),
            pl.BlockSpec((bm, n), lambda i, j: (i, 0)),
            pl.BlockSpec((n, bn), lambda i, j: (0, j)),
            ij, ij, ij,
        ],
        out_specs=[
            ij,
            pl.BlockSpec((bm, 1), lambda i, j: (i, 0)),
        ],
        out_shape=[
            jax.ShapeDtypeStruct((n, n), jnp.float32),
            jax.ShapeDtypeStruct((n, 1), jnp.float32),
        ],
        compiler_params=pltpu.CompilerParams(
            dimension_semantics=("parallel", "arbitrary")),
    )(cw_pad, t3, a, a, t2, t3)


def _colsum_body(tot_ref, dr_ref, dc_ref, s2_ref):
    m = tot_ref[...] * dr_ref[...] * dc_ref[...]
    s2_ref[...] = jnp.sum(m, axis=0, keepdims=True)


def _m_colsum(total, dis_r, dis_c, bn=512):
    n = total.shape[0]
    bn = min(bn, n)
    return pl.pallas_call(
        _colsum_body,
        grid=(n // bn,),
        in_specs=[
            pl.BlockSpec((n, bn), lambda j: (0, j)),
            pl.BlockSpec((n, 1), lambda j: (0, 0)),
            pl.BlockSpec((1, bn), lambda j: (0, j)),
        ],
        out_specs=pl.BlockSpec((1, bn), lambda j: (0, j)),
        out_shape=jax.ShapeDtypeStruct((1, n), jnp.float32),
        compiler_params=pltpu.CompilerParams(
            dimension_semantics=("parallel",)),
    )(total, dis_r, dis_c)


def _xside_body(tot_ref, dr_ref, dc_ref, x_ref, wt_ref, b_ref, p_ref,
                s2_ref, beta_ref, xs_ref, sc_ref):
    m = tot_ref[...] * dr_ref[...] * dc_ref[...]
    mx = jnp.dot(m, x_ref[...], preferred_element_type=jnp.float32)
    lin = jnp.dot(mx, wt_ref[...], preferred_element_type=jnp.float32) + b_ref[...]
    r = jnp.maximum(lin, 0.0)
    s1 = jnp.sum(r * p_ref[...], axis=1, keepdims=True)
    xs_ref[...] = r
    sc_ref[...] = beta_ref[0, 0] * s1 + beta_ref[0, 1] * s2_ref[...]


def _xside(total, dis_r, dis_c, x, wt, b2d, p2d, s2col, beta_pad, bm=1024):
    n = total.shape[0]
    d_in = x.shape[1]
    d_out = wt.shape[1]
    bm = min(bm, n)
    xs, sc = pl.pallas_call(
        _xside_body,
        grid=(n // bm,),
        in_specs=[
            pl.BlockSpec((bm, n), lambda i: (i, 0)),
            pl.BlockSpec((bm, 1), lambda i: (i, 0)),
            pl.BlockSpec((1, n), lambda i: (0, 0)),
            pl.BlockSpec((n, d_in), lambda i: (0, 0)),
            pl.BlockSpec((d_in, d_out), lambda i: (0, 0)),
            pl.BlockSpec((1, d_out), lambda i: (0, 0)),
            pl.BlockSpec((1, d_out), lambda i: (0, 0)),
            pl.BlockSpec((bm, 1), lambda i: (i, 0)),
            pl.BlockSpec((1, 128), lambda i: (0, 0)),
        ],
        out_specs=[
            pl.BlockSpec((bm, d_out), lambda i: (i, 0)),
            pl.BlockSpec((bm, 1), lambda i: (i, 0)),
        ],
        out_shape=[
            jax.ShapeDtypeStruct((n, d_out), jnp.float32),
            jax.ShapeDtypeStruct((n, 1), jnp.float32),
        ],
        compiler_params=pltpu.CompilerParams(
            dimension_semantics=("parallel",)),
    )(total, dis_r, dis_c, x, wt, b2d, p2d, s2col, beta_pad)
    return xs, sc


def _rank_body(sr_ref, sc_ref, r_ref, rank_ref, xs_ref, *, bm, n):
    i = pl.program_id(0)
    sr = sr_ref[...]
    sc = sc_ref[...]
    ig = i * bm + jax.lax.broadcasted_iota(jnp.int32, (bm, n), 0)
    jg = jax.lax.broadcasted_iota(jnp.int32, (bm, n), 1)
    gt = (sc > sr)
    eq_lt = (sc == sr) & (jg < ig)
    rank_ref[...] = jnp.sum((gt | eq_lt).astype(jnp.float32), axis=1,
                            keepdims=True)
    xs_ref[...] = r_ref[...] * sr


def _rank_scale(score_col, score_row, relu_out, bm=512):
    n = score_col.shape[0]
    d = relu_out.shape[1]
    bm = min(bm, n)
    return pl.pallas_call(
        functools.partial(_rank_body, bm=bm, n=n),
        grid=(n // bm,),
        in_specs=[
            pl.BlockSpec((bm, 1), lambda i: (i, 0)),
            pl.BlockSpec((1, n), lambda i: (0, 0)),
            pl.BlockSpec((bm, d), lambda i: (i, 0)),
        ],
        out_specs=[
            pl.BlockSpec((bm, 1), lambda i: (i, 0)),
            pl.BlockSpec((bm, d), lambda i: (i, 0)),
        ],
        out_shape=[
            jax.ShapeDtypeStruct((n, 1), jnp.float32),
            jax.ShapeDtypeStruct((n, d), jnp.float32),
        ],
        compiler_params=pltpu.CompilerParams(
            dimension_semantics=("parallel",)),
    )(score_col, score_row, relu_out)


def _perm_body(rank_ref, perm_ref, *, bq, n):
    r = pl.program_id(0)
    rg = (r * bq + jax.lax.broadcasted_iota(jnp.int32, (bq, n), 0)).astype(
        jnp.float32)
    ig = jax.lax.broadcasted_iota(jnp.int32, (bq, n), 1).astype(jnp.float32)
    eq = (rank_ref[...] == rg)
    perm_ref[...] = jnp.sum(jnp.where(eq, ig, 0.0), axis=1, keepdims=True)


def _perm_from_rank(rank_row, k, bq=512):
    n = rank_row.shape[1]
    bq = min(bq, k)
    return pl.pallas_call(
        functools.partial(_perm_body, bq=bq, n=n),
        grid=(k // bq,),
        in_specs=[pl.BlockSpec((1, n), lambda r: (0, 0))],
        out_specs=pl.BlockSpec((bq, 1), lambda r: (r, 0)),
        out_shape=jax.ShapeDtypeStruct((k, 1), jnp.float32),
        compiler_params=pltpu.CompilerParams(
            dimension_semantics=("parallel",)),
    )(rank_row)


_GR = 64


def _gather_body(pref, *refs):
    out_ref = refs[-1]
    for r in range(_GR):
        out_ref[r, 0, :] = refs[r][0, 0, :]


def _row_gather(src, perm):
    k = perm.shape[0]
    n, d = src.shape
    src3 = jnp.reshape(src, (n, 1, d))
    in_specs = [
        pl.BlockSpec((1, 1, d),
                     (lambda i, pref, r=r: (pref[i * _GR + r], 0, 0)))
        for r in range(_GR)
    ]
    grid_spec = pltpu.PrefetchScalarGridSpec(
        num_scalar_prefetch=1,
        grid=(k // _GR,),
        in_specs=in_specs,
        out_specs=pl.BlockSpec((_GR, 1, d), lambda i, pref: (i, 0, 0)),
    )
    out = pl.pallas_call(
        _gather_body,
        grid_spec=grid_spec,
        out_shape=jax.ShapeDtypeStruct((k, 1, d), jnp.float32),
    )(perm, *([src3] * _GR))
    return jnp.reshape(out, (k, d))


def _tr_body(src_ref, out_ref, *, binarize):
    t = src_ref[...].T
    if binarize:
        out_ref[...] = (t != 0.0).astype(jnp.float32)
    else:
        out_ref[...] = t


def _transpose(src, bm=256, binarize=False):
    m, n = src.shape
    bi = min(bm, n)
    bj = min(bm, m)
    return pl.pallas_call(
        functools.partial(_tr_body, binarize=binarize),
        grid=(n // bi, m // bj),
        in_specs=[pl.BlockSpec((bj, bi), lambda i, j: (j, i))],
        out_specs=pl.BlockSpec((bi, bj), lambda i, j: (i, j)),
        out_shape=jax.ShapeDtypeStruct((n, m), jnp.float32),
    )(src)


def _head_body(x_ref, w1_ref, b1_ref, w2_ref, b2_ref, w3_ref, b3_ref, o_ref):
    h = jnp.mean(x_ref[...], axis=0, keepdims=True)
    h = jnp.dot(h, w1_ref[...], preferred_element_type=jnp.float32) + b1_ref[...]
    h = jnp.maximum(h, 0.0)
    h = jnp.dot(h, w2_ref[...], preferred_element_type=jnp.float32) + b2_ref[...]
    h = jnp.maximum(h, 0.0)
    h = jnp.dot(h, w3_ref[...], preferred_element_type=jnp.float32) + b3_ref[...]
    o_ref[...] = h


def _head(x, w1, b1, w2, b2, w3, b3):
    n, d = x.shape
    return pl.pallas_call(
        _head_body,
        out_shape=jax.ShapeDtypeStruct((1, w3.shape[1]), jnp.float32),
    )(x, w1, b1, w2, b2, w3, b3)


def _pad_row(v, width=128):
    out = jnp.zeros((1, width), jnp.float32)
    return out.at[0, : v.shape[0]].set(v.astype(jnp.float32))


@jax.jit
def _run(x, edge_index, batch, params):
    n = x.shape[0]
    adj = jnp.zeros((n, n), jnp.float32).at[edge_index[1], edge_index[0]].set(1.0)

    for lp in params["layers"]:
        w = lp["w"].astype(jnp.float32)
        c1 = w[0] * w[1]
        cw_pad = _pad_row(jnp.concatenate([w, c1[None]]))
        beta_pad = _pad_row(lp["beta"])
        wt = jnp.transpose(lp["W"]).astype(jnp.float32)
        b2d = lp["b"].astype(jnp.float32).reshape(1, -1)
        p2d = lp["p"].astype(jnp.float32).reshape(1, -1)

        t2 = _chain_mm(cw_pad, adj, adj, pre_idx=5, post_idx=2)
        t3 = _chain_mm(cw_pad, t2, adj, pre_idx=None, post_idx=3)
        total, nnz = _total_nnz(cw_pad, adj, t2, t3)

        dis = jnp.maximum(nnz, 1.0) ** -0.5
        dis_c = dis.reshape(1, -1)
        s2 = _m_colsum(total, dis, dis_c)

        relu_out, pre = _xside(total, dis, dis_c, x, wt, b2d, p2d,
                               jnp.reshape(s2, (-1, 1)), beta_pad)
        score = jnp.tanh(pre)

        k = int(math.ceil(_RATIO * n))
        rank, xs_full = _rank_scale(score, jnp.reshape(score, (1, -1)),
                                    relu_out)
        perm_f = _perm_from_rank(jnp.reshape(rank, (1, -1)), k)
        perm = perm_f.astype(jnp.int32).reshape(-1)

        x = _row_gather(xs_full, perm)
        if lp is not params["layers"][-1]:
            g = _row_gather(total, perm)
            h = _transpose(g, binarize=True)
            adj = _row_gather(h, perm)
        n = k

    fc = params["fc"]
    w1 = jnp.transpose(fc[0]["W"]).astype(jnp.float32)
    b1 = fc[0]["b"].reshape(1, -1)
    w2 = jnp.zeros((128, 128), jnp.float32).at[:, :64].set(
        jnp.transpose(fc[1]["W"]))
    b2w = jnp.zeros((1, 128), jnp.float32).at[0, :64].set(fc[1]["b"])
    w3 = jnp.zeros((128, 128), jnp.float32).at[:64, :10].set(
        jnp.transpose(fc[2]["W"]))
    b3w = jnp.zeros((1, 128), jnp.float32).at[0, :10].set(fc[2]["b"])
    out = _head(x, w1, b1, w2, b2w, w3, b3w)
    return out[:, :10]


def kernel(x, edge_index, batch, params):
    return _run(x, edge_index, batch, params)
